# Initial kernel scaffold; baseline (speedup 1.0000x reference)
import jax
import jax.numpy as jnp
from jax.experimental import pallas as pl


def kernel(embeddings, edge_index):
    raise NotImplementedError("write your pallas kernel here")


# speedup vs baseline: 1.4947x; 1.4947x over previous
import functools

import jax
import jax.numpy as jnp
from jax import lax
from jax.experimental import pallas as pl
from jax.experimental.pallas import tpu as pltpu
from jax.experimental.pallas import tpu_sc as plsc

NC = 2
NS = 16
L = 16
NW = NC * NS

E = 320000
E_PAD = 327680
EPW = E_PAD // NW
C = 128
NCHUNK = EPW // C
D = 128


def _sc_partials(src_idx, dst_idx, emb):
    mesh = plsc.VectorSubcoreMesh(
        core_axis_name="c", subcore_axis_name="s", num_cores=NC,
        num_subcores=NS)

    @functools.partial(
        pl.kernel,
        out_type=jax.ShapeDtypeStruct((NW, L), jnp.float32),
        mesh=mesh,
        scratch_types=[
            pltpu.VMEM((C,), jnp.int32),
            pltpu.VMEM((C,), jnp.int32),
            pltpu.VMEM((C, D), jnp.float32),
            pltpu.VMEM((C, D), jnp.float32),
            pltpu.VMEM((L,), jnp.float32),
            pltpu.SemaphoreType.DMA,
        ],
    )
    def k(src_hbm, dst_hbm, emb_hbm, out_hbm, sidx, didx, srows, drows, accv,
          sem):
        wid = lax.axis_index("s") * NC + lax.axis_index("c")
        base = wid * EPW

        def chunk_body(c, grand):
            off = base + c * C
            pltpu.sync_copy(src_hbm.at[pl.ds(off, C)], sidx)
            pltpu.sync_copy(dst_hbm.at[pl.ds(off, C)], didx)
            cp_s = pltpu.async_copy(emb_hbm.at[sidx], srows, sem)
            cp_d = pltpu.async_copy(emb_hbm.at[didx], drows, sem)
            cp_s.wait()
            cp_d.wait()

            def row_body(r, acc):
                a = acc
                for j in range(D // L):
                    s = srows[r, pl.ds(j * L, L)]
                    d = drows[r, pl.ds(j * L, L)]
                    df = s - d
                    a = a + df * df
                return a

            ch = lax.fori_loop(0, C, row_body, jnp.zeros((L,), jnp.float32))
            return grand + ch

        grand = lax.fori_loop(0, NCHUNK, chunk_body,
                              jnp.zeros((L,), jnp.float32))
        accv[...] = grand
        pltpu.sync_copy(accv, out_hbm.at[wid])

    return k(src_idx, dst_idx, emb)


def kernel(embeddings, edge_index):
    idx = edge_index.astype(jnp.int32)
    pad = jnp.zeros((2, E_PAD - E), jnp.int32)
    idx = jnp.concatenate([idx, pad], axis=1)
    partials = _sc_partials(idx[0], idx[1], embeddings)
    return (0.1 / (E * D)) * jnp.sum(partials)


# speedup vs baseline: 1.6996x; 1.1371x over previous
import functools

import jax
import jax.numpy as jnp
from jax import lax
from jax.experimental import pallas as pl
from jax.experimental.pallas import tpu as pltpu
from jax.experimental.pallas import tpu_sc as plsc

NC = 2
NS = 16
L = 16
NW = NC * NS

E = 320000
E_PAD = 327680
EPW = E_PAD // NW
C = 128
NCHUNK = EPW // C
D = 128
NBUF = 2


def _sc_partials(src_idx, dst_idx, emb):
    mesh = plsc.VectorSubcoreMesh(
        core_axis_name="c", subcore_axis_name="s", num_cores=NC,
        num_subcores=NS)

    @functools.partial(
        pl.kernel,
        out_type=jax.ShapeDtypeStruct((NW, L), jnp.float32),
        mesh=mesh,
        scratch_types=[
            pltpu.VMEM((NCHUNK, C), jnp.int32),
            pltpu.VMEM((NCHUNK, C), jnp.int32),
            pltpu.VMEM((NBUF, C, D), jnp.float32),
            pltpu.VMEM((NBUF, C, D), jnp.float32),
            pltpu.VMEM((L,), jnp.float32),
            pltpu.SemaphoreType.DMA,
            pltpu.SemaphoreType.DMA,
        ],
    )
    def k(src_hbm, dst_hbm, emb_hbm, out_hbm, sidx, didx, srows, drows, accv,
          sem0, sem1):
        wid = lax.axis_index("s") * NC + lax.axis_index("c")
        sems = (sem0, sem1)

        pltpu.sync_copy(src_hbm.at[wid], sidx)
        pltpu.sync_copy(dst_hbm.at[wid], didx)

        def issue(b, c):
            pltpu.async_copy(emb_hbm.at[sidx.at[c]], srows.at[b], sems[b])
            pltpu.async_copy(emb_hbm.at[didx.at[c]], drows.at[b], sems[b])

        def drain(b, c):
            pltpu.make_async_copy(
                emb_hbm.at[sidx.at[c]], srows.at[b], sems[b]).wait()
            pltpu.make_async_copy(
                emb_hbm.at[didx.at[c]], drows.at[b], sems[b]).wait()

        for b in range(NBUF):
            issue(b, b)

        @pl.loop(0, NCHUNK, step=NBUF,
                 init_carry=jnp.zeros((L,), jnp.float32))
        def outer(c, acc):
            for b in range(NBUF):
                cur = c + b
                drain(b, cur)

                def row_body(r, a):
                    for j in range(D // L):
                        s = srows[b, r, pl.ds(j * L, L)]
                        d = drows[b, r, pl.ds(j * L, L)]
                        df = s - d
                        a = a + df * df
                    return a

                ch = lax.fori_loop(0, C, row_body,
                                   jnp.zeros((L,), jnp.float32), unroll=2)
                acc = acc + ch

                nxt = cur + NBUF

                @pl.when(nxt < NCHUNK)
                def _():
                    issue(b, nxt)

            return acc

        accv[...] = outer
        pltpu.sync_copy(accv, out_hbm.at[wid])

    return k(src_idx, dst_idx, emb)


def kernel(embeddings, edge_index):
    idx = edge_index.astype(jnp.int32)
    pad = jnp.zeros((2, E_PAD - E), jnp.int32)
    idx = jnp.concatenate([idx, pad], axis=1)
    idx = idx.reshape(2, NW, NCHUNK, C)
    partials = _sc_partials(idx[0], idx[1], embeddings)
    return (0.1 / (E * D)) * jnp.sum(partials)
